# baseline (speedup 1.0000x reference)
import functools

import jax
import jax.numpy as jnp
from jax import lax
from jax.experimental import pallas as pl
from jax.experimental.pallas import tpu as pltpu
from jax.experimental.pallas import tpu_sc as plsc

N = 10000
E = 320000
D = 128
NC = 2
NS = 16
NW = NC * NS
EP = 327680
DCHUNK = 128
DCH = (EP // NW) // DCHUNK
CHUNK = 80
N_CHUNKS = (EP // NW) // CHUNK
GR = 64
NT = 10080
IO = 80
IDXR = 4
RING = 3

_MESH = plsc.VectorSubcoreMesh(
    core_axis_name="c", subcore_axis_name="s", num_cores=NC, num_subcores=NS)


def _strided_copy(n_chunks, sid, body):
  for k in range(-(-n_chunks // NS)):
    j = k * NS + sid

    @pl.when(j < n_chunks)
    def _():
      body(j)


@functools.partial(
    pl.kernel,
    out_type=jax.ShapeDtypeStruct((NC, N, D), jnp.float32),
    mesh=_MESH,
    scratch_types=[
        pltpu.VMEM_SHARED((NT, D), jnp.float32),
        pltpu.VMEM((DCH, DCHUNK), jnp.int32),
        pltpu.VMEM((DCHUNK, D), jnp.float32),
        pltpu.SemaphoreType.DMA,
    ],
)
def _deg_kernel(dst3d_hbm, zrows_hbm, ones_hbm, deg_out,
                deg_sp, didx_all, ones_v, sem):
  cid = lax.axis_index("c")
  sid = lax.axis_index("s")
  _strided_copy(N // IO, sid,
                lambda j: pltpu.sync_copy(zrows_hbm,
                                          deg_sp.at[pl.ds(j * IO, IO)]))
  pltpu.sync_copy(ones_hbm, ones_v)
  wid = cid * NS + sid
  pltpu.sync_copy(dst3d_hbm.at[wid], didx_all)
  plsc.subcore_barrier()

  FD = 5

  def blk(k, _):
    base = k * FD
    for j in range(FD):
      pltpu.async_copy(ones_v, deg_sp.at[didx_all.at[base + j]], sem, add=True)
    for j in range(FD):
      pltpu.make_async_copy(ones_v, deg_sp.at[didx_all.at[base + j]],
                            sem).wait()
    return ()

  lax.fori_loop(0, DCH // FD, blk, ())
  plsc.subcore_barrier()
  _strided_copy(N // IO, sid,
                lambda j: pltpu.sync_copy(
                    deg_sp.at[pl.ds(j * IO, IO)],
                    deg_out.at[cid, pl.ds(j * IO, IO)]))


@functools.partial(
    pl.kernel,
    out_type=jax.ShapeDtypeStruct((NC, N, D), jnp.float32),
    mesh=_MESH,
    scratch_types=[
        pltpu.VMEM_SHARED((NT, D), jnp.float32),
        [pltpu.VMEM((CHUNK,), jnp.int32)] * IDXR,
        [pltpu.VMEM((CHUNK,), jnp.int32)] * IDXR,
        [pltpu.VMEM((CHUNK, D), jnp.float32)] * RING,
        [pltpu.SemaphoreType.DMA] * IDXR,
        [pltpu.SemaphoreType.DMA] * RING,
        pltpu.SemaphoreType.DMA,
    ],
)
def _agg_kernel(y_hbm, src_hbm, dst_hbm, zrows_hbm, z_out,
                z_sp, sidx, didx, rows, semi, semg, sems):
  cid = lax.axis_index("c")
  sid = lax.axis_index("s")
  _strided_copy(N // IO, sid,
                lambda j: pltpu.sync_copy(zrows_hbm,
                                          z_sp.at[pl.ds(j * IO, IO)]))
  wid = cid * NS + sid
  base0 = wid * N_CHUNKS

  def load_idx(j, s, sync):
    hs = src_hbm.at[pl.ds((base0 + j) * CHUNK, CHUNK)]
    hd = dst_hbm.at[pl.ds((base0 + j) * CHUNK, CHUNK)]
    if sync:
      pltpu.sync_copy(hs, sidx[s])
      pltpu.sync_copy(hd, didx[s])
    else:
      pltpu.async_copy(hs, sidx[s], semi[s])
      pltpu.async_copy(hd, didx[s], semi[s])

  def wait_idx(j, s):
    eb = (base0 + j) * CHUNK
    pltpu.make_async_copy(src_hbm.at[pl.ds(eb, CHUNK)], sidx[s],
                          semi[s]).wait()
    pltpu.make_async_copy(dst_hbm.at[pl.ds(eb, CHUNK)], didx[s],
                          semi[s]).wait()

  def drain_scatter():
    pltpu.make_async_copy(rows[0], z_sp.at[didx[0]], sems).wait()

  plsc.subcore_barrier()

  load_idx(0, 0, True)
  pltpu.async_copy(y_hbm.at[sidx[0]], rows[0], semg[0])
  load_idx(1, 1, False)
  load_idx(2, 2, False)

  def handle(j, si, ri, drain):
    pltpu.make_async_copy(y_hbm.at[sidx[si]], rows[ri], semg[ri]).wait()
    pltpu.async_copy(rows[ri], z_sp.at[didx[si]], sems, add=True)
    if drain:
      drain_scatter()

    @pl.when(j + 1 < N_CHUNKS)
    def _():
      wait_idx(j + 1, (si + 1) % IDXR)
      pltpu.async_copy(y_hbm.at[sidx[(si + 1) % IDXR]],
                       rows[(ri + 1) % RING], semg[(ri + 1) % RING])

    @pl.when(j + 3 < N_CHUNKS)
    def _():
      load_idx(j + 3, (si + 3) % IDXR, False)

  def step(k, _):
    base = 12 * k
    for u in range(12):
      handle(base + u, u % IDXR, u % RING, True)
    return ()

  for u in range(12):
    handle(u, u % IDXR, u % RING, u > 0)
  lax.fori_loop(1, N_CHUNKS // 12, step, ())
  for u in range(120, N_CHUNKS):
    handle(u, u % IDXR, u % RING, True)
  drain_scatter()
  plsc.subcore_barrier()
  _strided_copy(N // IO, sid,
                lambda j: pltpu.sync_copy(
                    z_sp.at[pl.ds(j * IO, IO)],
                    z_out.at[cid, pl.ds(j * IO, IO)]))


_ROWS_TC = 1000


def _xw_body(x_ref, w_ref, o_ref):
  o_ref[...] = jnp.dot(x_ref[...], w_ref[...],
                       preferred_element_type=jnp.float32)


def _scale_body(deg_ref, xw_ref, y_ref, dis_ref):
  deg = deg_ref[0, :, 0:1] + deg_ref[1, :, 0:1] + 1.0
  dis = lax.rsqrt(deg)
  dis_ref[...] = dis
  y_ref[...] = dis * xw_ref[...]


def _final_body(z_ref, y_ref, dis_ref, b_ref, o_ref):
  o_ref[...] = dis_ref[...] * (z_ref[0] + z_ref[1] + y_ref[...]) + b_ref[...]


def _tc_xw(x, W):
  grid = (N // _ROWS_TC,)
  return pl.pallas_call(
      _xw_body,
      grid=grid,
      in_specs=[
          pl.BlockSpec((_ROWS_TC, D), lambda i: (i, 0)),
          pl.BlockSpec((D, D), lambda i: (0, 0)),
      ],
      out_specs=pl.BlockSpec((_ROWS_TC, D), lambda i: (i, 0)),
      out_shape=jax.ShapeDtypeStruct((N, D), jnp.float32),
  )(x, W)


def _tc_scale(degparts, xw):
  grid = (N // _ROWS_TC,)
  return pl.pallas_call(
      _scale_body,
      grid=grid,
      in_specs=[
          pl.BlockSpec((NC, _ROWS_TC, D), lambda i: (0, i, 0)),
          pl.BlockSpec((_ROWS_TC, D), lambda i: (i, 0)),
      ],
      out_specs=[
          pl.BlockSpec((_ROWS_TC, D), lambda i: (i, 0)),
          pl.BlockSpec((_ROWS_TC, 1), lambda i: (i, 0)),
      ],
      out_shape=[
          jax.ShapeDtypeStruct((N, D), jnp.float32),
          jax.ShapeDtypeStruct((N, 1), jnp.float32),
      ],
  )(degparts, xw)


def _tc_final(zparts, y, dis, b2):
  grid = (N // _ROWS_TC,)
  return pl.pallas_call(
      _final_body,
      grid=grid,
      in_specs=[
          pl.BlockSpec((NC, _ROWS_TC, D), lambda i: (0, i, 0)),
          pl.BlockSpec((_ROWS_TC, D), lambda i: (i, 0)),
          pl.BlockSpec((_ROWS_TC, 1), lambda i: (i, 0)),
          pl.BlockSpec((1, D), lambda i: (0, 0)),
      ],
      out_specs=pl.BlockSpec((_ROWS_TC, D), lambda i: (i, 0)),
      out_shape=jax.ShapeDtypeStruct((N, D), jnp.float32),
  )(zparts, y, dis, b2)


def kernel(x, edge_index, W, b):
  ei = edge_index.astype(jnp.int32)
  pad = EP - E
  pads = jnp.concatenate(
      [jnp.zeros((1, pad), jnp.int32),
       (N + jnp.arange(pad, dtype=jnp.int32) % GR).reshape(1, pad)], axis=0)
  ei = jnp.concatenate([ei, pads], axis=1)
  src = ei[0]
  dst = ei[1]
  dst3d = dst.reshape(NW, DCH, DCHUNK)
  ones = jnp.ones((DCHUNK, D), jnp.float32)
  zrows = jnp.zeros((IO, D), jnp.float32)

  xw = _tc_xw(x, W)
  degparts = _deg_kernel(dst3d, zrows, ones)
  y, dis = _tc_scale(degparts, xw)
  zparts = _agg_kernel(y, src, dst, zrows)
  return _tc_final(zparts, y, dis, b.reshape(1, D))


# speedup vs baseline: 2.3798x; 2.3798x over previous
import functools

import jax
import jax.numpy as jnp
from jax import lax
from jax.experimental import pallas as pl
from jax.experimental.pallas import tpu as pltpu
from jax.experimental.pallas import tpu_sc as plsc

N = 10000
E = 320000
D = 128
NC = 2
NS = 16
NW = NC * NS
EP = 327680
DCHUNK = 128
DCH = (EP // NW) // DCHUNK
CHUNK = 80
N_CHUNKS = (EP // NW) // CHUNK
GR = 64
NT = 10080
IO = 80
IDXR = 4
RING = 3

_MESH = plsc.VectorSubcoreMesh(
    core_axis_name="c", subcore_axis_name="s", num_cores=NC, num_subcores=NS)


def _strided_copy(n_chunks, sid, body):
  for k in range(-(-n_chunks // NS)):
    j = k * NS + sid

    @pl.when(j < n_chunks)
    def _():
      body(j)


@functools.partial(
    pl.kernel,
    out_type=jax.ShapeDtypeStruct((NC, N, D), jnp.float32),
    mesh=_MESH,
    scratch_types=[
        pltpu.VMEM_SHARED((NT, D), jnp.float32),
        pltpu.VMEM((DCH, DCHUNK), jnp.int32),
        pltpu.VMEM((DCHUNK, D), jnp.float32),
        pltpu.SemaphoreType.DMA,
    ],
)
def _deg_kernel(dst3d_hbm, zrows_hbm, ones_hbm, deg_out,
                deg_sp, didx_all, ones_v, sem):
  cid = lax.axis_index("c")
  sid = lax.axis_index("s")
  _strided_copy(N // IO, sid,
                lambda j: pltpu.sync_copy(zrows_hbm,
                                          deg_sp.at[pl.ds(j * IO, IO)]))
  pltpu.sync_copy(ones_hbm, ones_v)
  wid = cid * NS + sid
  pltpu.sync_copy(dst3d_hbm.at[wid], didx_all)
  plsc.subcore_barrier()

  FD = 5

  def blk(k, _):
    base = k * FD
    for j in range(FD):
      pltpu.async_copy(ones_v, deg_sp.at[didx_all.at[base + j]], sem, add=True)
    for j in range(FD):
      pltpu.make_async_copy(ones_v, deg_sp.at[didx_all.at[base + j]],
                            sem).wait()
    return ()

  lax.fori_loop(0, DCH // FD, blk, ())
  plsc.subcore_barrier()
  _strided_copy(N // IO, sid,
                lambda j: pltpu.sync_copy(
                    deg_sp.at[pl.ds(j * IO, IO)],
                    deg_out.at[cid, pl.ds(j * IO, IO)]))


@functools.partial(
    pl.kernel,
    out_type=jax.ShapeDtypeStruct((NC, N, D), jnp.float32),
    mesh=_MESH,
    scratch_types=[
        pltpu.VMEM_SHARED((NT, D), jnp.float32),
        [pltpu.VMEM((CHUNK,), jnp.int32)] * IDXR,
        [pltpu.VMEM((CHUNK,), jnp.int32)] * IDXR,
        [pltpu.VMEM((CHUNK, D), jnp.float32)] * RING,
        [pltpu.SemaphoreType.DMA] * IDXR,
        [pltpu.SemaphoreType.DMA] * RING,
        pltpu.SemaphoreType.DMA,
    ],
)
def _agg_kernel(y_hbm, src_hbm, dst_hbm, zrows_hbm, z_out,
                z_sp, sidx, didx, rows, semi, semg, sems):
  cid = lax.axis_index("c")
  sid = lax.axis_index("s")
  _strided_copy(N // IO, sid,
                lambda j: pltpu.sync_copy(zrows_hbm,
                                          z_sp.at[pl.ds(j * IO, IO)]))
  wid = cid * NS + sid
  base0 = wid * N_CHUNKS

  def load_idx(j, s, sync):
    hs = src_hbm.at[pl.ds((base0 + j) * CHUNK, CHUNK)]
    hd = dst_hbm.at[pl.ds((base0 + j) * CHUNK, CHUNK)]
    if sync:
      pltpu.sync_copy(hs, sidx[s])
      pltpu.sync_copy(hd, didx[s])
    else:
      pltpu.async_copy(hs, sidx[s], semi[s])
      pltpu.async_copy(hd, didx[s], semi[s])

  def wait_idx(j, s):
    eb = (base0 + j) * CHUNK
    pltpu.make_async_copy(src_hbm.at[pl.ds(eb, CHUNK)], sidx[s],
                          semi[s]).wait()
    pltpu.make_async_copy(dst_hbm.at[pl.ds(eb, CHUNK)], didx[s],
                          semi[s]).wait()

  def drain_scatter():
    pltpu.make_async_copy(rows[0], z_sp.at[didx[0]], sems).wait()

  plsc.subcore_barrier()

  load_idx(0, 0, True)
  pltpu.async_copy(y_hbm.at[sidx[0]], rows[0], semg[0])
  load_idx(1, 1, False)
  load_idx(2, 2, False)

  def handle(j, si, ri, drain):
    @pl.when(j + 1 < N_CHUNKS)
    def _():
      wait_idx(j + 1, (si + 1) % IDXR)
      pltpu.async_copy(y_hbm.at[sidx[(si + 1) % IDXR]],
                       rows[(ri + 1) % RING], semg[(ri + 1) % RING])

    pltpu.make_async_copy(y_hbm.at[sidx[si]], rows[ri], semg[ri]).wait()
    pltpu.async_copy(rows[ri], z_sp.at[didx[si]], sems, add=True)
    if drain:
      drain_scatter()

    @pl.when(j + 3 < N_CHUNKS)
    def _():
      load_idx(j + 3, (si + 3) % IDXR, False)

  def step(k, _):
    base = 12 * k
    for u in range(12):
      handle(base + u, u % IDXR, u % RING, True)
    return ()

  for u in range(12):
    handle(u, u % IDXR, u % RING, u > 0)
  lax.fori_loop(1, N_CHUNKS // 12, step, ())
  for u in range(120, N_CHUNKS):
    handle(u, u % IDXR, u % RING, True)
  drain_scatter()
  plsc.subcore_barrier()
  _strided_copy(N // IO, sid,
                lambda j: pltpu.sync_copy(
                    z_sp.at[pl.ds(j * IO, IO)],
                    z_out.at[cid, pl.ds(j * IO, IO)]))


_ROWS_TC = 1000


def _xw_body(x_ref, w_ref, o_ref):
  o_ref[...] = jnp.dot(x_ref[...], w_ref[...],
                       preferred_element_type=jnp.float32)


def _scale_body(deg_ref, xw_ref, y_ref, dis_ref):
  deg = deg_ref[0, :, 0:1] + deg_ref[1, :, 0:1] + 1.0
  dis = lax.rsqrt(deg)
  dis_ref[...] = dis
  y_ref[...] = dis * xw_ref[...]


def _final_body(z_ref, y_ref, dis_ref, b_ref, o_ref):
  o_ref[...] = dis_ref[...] * (z_ref[0] + z_ref[1] + y_ref[...]) + b_ref[...]


def _tc_xw(x, W):
  grid = (N // _ROWS_TC,)
  return pl.pallas_call(
      _xw_body,
      grid=grid,
      in_specs=[
          pl.BlockSpec((_ROWS_TC, D), lambda i: (i, 0)),
          pl.BlockSpec((D, D), lambda i: (0, 0)),
      ],
      out_specs=pl.BlockSpec((_ROWS_TC, D), lambda i: (i, 0)),
      out_shape=jax.ShapeDtypeStruct((N, D), jnp.float32),
  )(x, W)


def _tc_scale(degparts, xw):
  grid = (N // _ROWS_TC,)
  return pl.pallas_call(
      _scale_body,
      grid=grid,
      in_specs=[
          pl.BlockSpec((NC, _ROWS_TC, D), lambda i: (0, i, 0)),
          pl.BlockSpec((_ROWS_TC, D), lambda i: (i, 0)),
      ],
      out_specs=[
          pl.BlockSpec((_ROWS_TC, D), lambda i: (i, 0)),
          pl.BlockSpec((_ROWS_TC, 1), lambda i: (i, 0)),
      ],
      out_shape=[
          jax.ShapeDtypeStruct((N, D), jnp.float32),
          jax.ShapeDtypeStruct((N, 1), jnp.float32),
      ],
  )(degparts, xw)


def _tc_final(zparts, y, dis, b2):
  grid = (N // _ROWS_TC,)
  return pl.pallas_call(
      _final_body,
      grid=grid,
      in_specs=[
          pl.BlockSpec((NC, _ROWS_TC, D), lambda i: (0, i, 0)),
          pl.BlockSpec((_ROWS_TC, D), lambda i: (i, 0)),
          pl.BlockSpec((_ROWS_TC, 1), lambda i: (i, 0)),
          pl.BlockSpec((1, D), lambda i: (0, 0)),
      ],
      out_specs=pl.BlockSpec((_ROWS_TC, D), lambda i: (i, 0)),
      out_shape=jax.ShapeDtypeStruct((N, D), jnp.float32),
  )(zparts, y, dis, b2)


def kernel(x, edge_index, W, b):
  ei = edge_index.astype(jnp.int32)
  pad = EP - E
  spread = jnp.arange(pad, dtype=jnp.int32) % GR
  pads = jnp.concatenate(
      [spread.reshape(1, pad), (N + spread).reshape(1, pad)], axis=0)
  ei = jnp.concatenate([ei, pads], axis=1)
  src = ei[0]
  dst = ei[1]
  dst3d = dst.reshape(NW, DCH, DCHUNK)
  ones = jnp.ones((DCHUNK, D), jnp.float32)
  zrows = jnp.zeros((IO, D), jnp.float32)

  xw = _tc_xw(x, W)
  degparts = _deg_kernel(dst3d, zrows, ones)
  y, dis = _tc_scale(degparts, xw)
  zparts = _agg_kernel(y, src, dst, zrows)
  return _tc_final(zparts, y, dis, b.reshape(1, D))


# speedup vs baseline: 2.4204x; 1.0171x over previous
import functools

import jax
import jax.numpy as jnp
from jax import lax
from jax.experimental import pallas as pl
from jax.experimental.pallas import tpu as pltpu
from jax.experimental.pallas import tpu_sc as plsc

N = 10000
E = 320000
D = 128
NC = 2
NS = 16
NW = NC * NS
EP = 327680
DCHUNK = 128
DCH = (EP // NW) // DCHUNK
CHUNK = 80
N_CHUNKS = (EP // NW) // CHUNK
GR = 64
NT = 10080
IO = 80
IDXR = 4
RING = 3

_MESH = plsc.VectorSubcoreMesh(
    core_axis_name="c", subcore_axis_name="s", num_cores=NC, num_subcores=NS)


def _strided_copy(n_chunks, sid, body):
  for k in range(-(-n_chunks // NS)):
    j = k * NS + sid

    @pl.when(j < n_chunks)
    def _():
      body(j)


@functools.partial(
    pl.kernel,
    out_type=jax.ShapeDtypeStruct((NC, N, D), jnp.float32),
    mesh=_MESH,
    scratch_types=[
        pltpu.VMEM_SHARED((NT, D), jnp.float32),
        pltpu.VMEM((DCH, DCHUNK), jnp.int32),
        pltpu.VMEM((DCHUNK, D), jnp.float32),
        pltpu.SemaphoreType.DMA,
    ],
)
def _deg_kernel(dst3d_hbm, zrows_hbm, ones_hbm, deg_out,
                deg_sp, didx_all, ones_v, sem):
  cid = lax.axis_index("c")
  sid = lax.axis_index("s")
  _strided_copy(N // IO, sid,
                lambda j: pltpu.sync_copy(zrows_hbm,
                                          deg_sp.at[pl.ds(j * IO, IO)]))
  pltpu.sync_copy(ones_hbm, ones_v)
  wid = cid * NS + sid
  pltpu.sync_copy(dst3d_hbm.at[wid], didx_all)
  plsc.subcore_barrier()

  FD = 5

  def blk(k, _):
    base = k * FD
    for j in range(FD):
      pltpu.async_copy(ones_v, deg_sp.at[didx_all.at[base + j]], sem, add=True)
    for j in range(FD):
      pltpu.make_async_copy(ones_v, deg_sp.at[didx_all.at[base + j]],
                            sem).wait()
    return ()

  lax.fori_loop(0, DCH // FD, blk, ())
  plsc.subcore_barrier()
  _strided_copy(N // IO, sid,
                lambda j: pltpu.sync_copy(
                    deg_sp.at[pl.ds(j * IO, IO)],
                    deg_out.at[cid, pl.ds(j * IO, IO)]))


@functools.partial(
    pl.kernel,
    out_type=jax.ShapeDtypeStruct((NC, N, D), jnp.float32),
    mesh=_MESH,
    scratch_types=[
        pltpu.VMEM_SHARED((NT, D), jnp.float32),
        [pltpu.VMEM((CHUNK,), jnp.int32)] * IDXR,
        [pltpu.VMEM((CHUNK,), jnp.int32)] * IDXR,
        [pltpu.VMEM((CHUNK, D), jnp.float32)] * RING,
        [pltpu.SemaphoreType.DMA] * IDXR,
        [pltpu.SemaphoreType.DMA] * RING,
        pltpu.SemaphoreType.DMA,
    ],
)
def _agg_kernel(y_hbm, src_hbm, dst_hbm, zrows_hbm, z_out,
                z_sp, sidx, didx, rows, semi, semg, sems):
  cid = lax.axis_index("c")
  sid = lax.axis_index("s")
  _strided_copy(N // IO, sid,
                lambda j: pltpu.sync_copy(zrows_hbm,
                                          z_sp.at[pl.ds(j * IO, IO)]))
  wid = cid * NS + sid
  base0 = wid * N_CHUNKS

  def load_idx(j, s, sync):
    hs = src_hbm.at[pl.ds((base0 + j) * CHUNK, CHUNK)]
    hd = dst_hbm.at[pl.ds((base0 + j) * CHUNK, CHUNK)]
    if sync:
      pltpu.sync_copy(hs, sidx[s])
      pltpu.sync_copy(hd, didx[s])
    else:
      pltpu.async_copy(hs, sidx[s], semi[s])
      pltpu.async_copy(hd, didx[s], semi[s])

  def wait_idx(j, s):
    eb = (base0 + j) * CHUNK
    pltpu.make_async_copy(src_hbm.at[pl.ds(eb, CHUNK)], sidx[s],
                          semi[s]).wait()
    pltpu.make_async_copy(dst_hbm.at[pl.ds(eb, CHUNK)], didx[s],
                          semi[s]).wait()

  def drain_scatter():
    pltpu.make_async_copy(rows[0], z_sp.at[didx[0]], sems).wait()

  plsc.subcore_barrier()

  load_idx(0, 0, True)
  pltpu.async_copy(y_hbm.at[sidx[0]], rows[0], semg[0])
  load_idx(1, 1, False)
  load_idx(2, 2, False)

  def handle(j, si, ri, drain):
    @pl.when(j + 1 < N_CHUNKS)
    def _():
      wait_idx(j + 1, (si + 1) % IDXR)
      pltpu.async_copy(y_hbm.at[sidx[(si + 1) % IDXR]],
                       rows[(ri + 1) % RING], semg[(ri + 1) % RING])

    pltpu.make_async_copy(y_hbm.at[sidx[si]], rows[ri], semg[ri]).wait()
    pltpu.async_copy(rows[ri], z_sp.at[didx[si]], sems, add=True)
    if drain:
      drain_scatter()

    @pl.when(j + 3 < N_CHUNKS)
    def _():
      load_idx(j + 3, (si + 3) % IDXR, False)

  def step(k, _):
    base = 12 * k
    for u in range(12):
      handle(base + u, u % IDXR, u % RING, True)
    return ()

  for u in range(12):
    handle(u, u % IDXR, u % RING, u > 0)
  lax.fori_loop(1, N_CHUNKS // 12, step, ())
  for u in range(120, N_CHUNKS):
    handle(u, u % IDXR, u % RING, True)
  drain_scatter()
  plsc.subcore_barrier()
  _strided_copy(N // IO, sid,
                lambda j: pltpu.sync_copy(
                    z_sp.at[pl.ds(j * IO, IO)],
                    z_out.at[cid, pl.ds(j * IO, IO)]))


_ROWS_TC = 2000


def _xw_body(x_ref, w_ref, o_ref):
  o_ref[...] = jnp.dot(x_ref[...], w_ref[...],
                       preferred_element_type=jnp.float32)


def _scale_body(deg_ref, xw_ref, y_ref, dis_ref):
  deg = deg_ref[0, :, 0:1] + deg_ref[1, :, 0:1] + 1.0
  dis = lax.rsqrt(deg)
  dis_ref[...] = dis
  y_ref[...] = dis * xw_ref[...]


def _final_body(z_ref, y_ref, dis_ref, b_ref, o_ref):
  o_ref[...] = dis_ref[...] * (z_ref[0] + z_ref[1] + y_ref[...]) + b_ref[...]


def _tc_xw(x, W):
  grid = (N // _ROWS_TC,)
  return pl.pallas_call(
      _xw_body,
      grid=grid,
      in_specs=[
          pl.BlockSpec((_ROWS_TC, D), lambda i: (i, 0)),
          pl.BlockSpec((D, D), lambda i: (0, 0)),
      ],
      out_specs=pl.BlockSpec((_ROWS_TC, D), lambda i: (i, 0)),
      out_shape=jax.ShapeDtypeStruct((N, D), jnp.float32),
  )(x, W)


def _tc_scale(degparts, xw):
  grid = (N // _ROWS_TC,)
  return pl.pallas_call(
      _scale_body,
      grid=grid,
      in_specs=[
          pl.BlockSpec((NC, _ROWS_TC, D), lambda i: (0, i, 0)),
          pl.BlockSpec((_ROWS_TC, D), lambda i: (i, 0)),
      ],
      out_specs=[
          pl.BlockSpec((_ROWS_TC, D), lambda i: (i, 0)),
          pl.BlockSpec((_ROWS_TC, 1), lambda i: (i, 0)),
      ],
      out_shape=[
          jax.ShapeDtypeStruct((N, D), jnp.float32),
          jax.ShapeDtypeStruct((N, 1), jnp.float32),
      ],
  )(degparts, xw)


def _tc_final(zparts, y, dis, b2):
  grid = (N // _ROWS_TC,)
  return pl.pallas_call(
      _final_body,
      grid=grid,
      in_specs=[
          pl.BlockSpec((NC, _ROWS_TC, D), lambda i: (0, i, 0)),
          pl.BlockSpec((_ROWS_TC, D), lambda i: (i, 0)),
          pl.BlockSpec((_ROWS_TC, 1), lambda i: (i, 0)),
          pl.BlockSpec((1, D), lambda i: (0, 0)),
      ],
      out_specs=pl.BlockSpec((_ROWS_TC, D), lambda i: (i, 0)),
      out_shape=jax.ShapeDtypeStruct((N, D), jnp.float32),
  )(zparts, y, dis, b2)


def kernel(x, edge_index, W, b):
  ei = edge_index.astype(jnp.int32)
  pad = EP - E
  spread = jnp.arange(pad, dtype=jnp.int32) % GR
  pads = jnp.concatenate(
      [spread.reshape(1, pad), (N + spread).reshape(1, pad)], axis=0)
  ei = jnp.concatenate([ei, pads], axis=1)
  src = ei[0]
  dst = ei[1]
  dst3d = dst.reshape(NW, DCH, DCHUNK)
  ones = jnp.ones((DCHUNK, D), jnp.float32)
  zrows = jnp.zeros((IO, D), jnp.float32)

  xw = _tc_xw(x, W)
  degparts = _deg_kernel(dst3d, zrows, ones)
  y, dis = _tc_scale(degparts, xw)
  zparts = _agg_kernel(y, src, dst, zrows)
  return _tc_final(zparts, y, dis, b.reshape(1, D))
